# baseline (speedup 1.0000x reference)
import functools

import jax
import jax.numpy as jnp
from jax import lax
from jax.experimental import pallas as pl
from jax.experimental.pallas import tpu as pltpu
from jax.experimental.pallas import tpu_sc as plsc

_F = 26
_NB = 10000
_BATCH = 16384
_LANES = 16
_PERIOD = 208
_KVECS = _PERIOD // _LANES
_NC = 2
_NS = 16
_NW = _NC * _NS
_E = _BATCH * _F // _NW
_GROUPS = _E // _PERIOD


@functools.partial(
    pl.kernel,
    mesh=plsc.VectorSubcoreMesh(core_axis_name="c", subcore_axis_name="s"),
    out_type=[
        jax.ShapeDtypeStruct((_BATCH * _F,), jnp.int32),
        jax.ShapeDtypeStruct((_BATCH * _F,), jnp.int32),
    ],
    scratch_types=[
        pltpu.VMEM((_E,), jnp.int32),
        pltpu.VMEM((_E,), jnp.int32),
        pltpu.VMEM((_E,), jnp.int32),
        pltpu.VMEM((_KVECS, _LANES), jnp.int32),
        pltpu.VMEM((_KVECS, _LANES), jnp.int32),
    ],
    compiler_params=pltpu.CompilerParams(needs_layout_passes=False),
)
def _tokenize(feats_hbm, pidx_hbm, pbase_hbm, ids_hbm, mask_hbm,
              feats_v, ids_v, mask_v, pidx_v, pbase_v):
    wid = lax.axis_index("s") * _NC + lax.axis_index("c")
    base = wid * _E
    pltpu.sync_copy(feats_hbm.at[pl.ds(base, _E)], feats_v)
    pltpu.sync_copy(pidx_hbm, pidx_v)
    pltpu.sync_copy(pbase_hbm, pbase_v)

    pidx = [pidx_v[kk, :] for kk in range(_KVECS)]
    pbase = [pbase_v[kk, :] for kk in range(_KVECS)]

    def group(g, carry):
        goff = g * _PERIOD
        for kk in range(_KVECS):
            vals = plsc.load_gather(feats_v, [pidx[kk] + goff])
            bucket = lax.rem(vals, _NB) + 1
            vid = pbase[kk] + bucket
            valid = vals > 0
            off = goff + kk * _LANES
            ids_v[pl.ds(off, _LANES)] = jnp.where(valid, vid, 0)
            mask_v[pl.ds(off, _LANES)] = jnp.where(valid, 1, 0)
        return carry

    lax.fori_loop(0, _GROUPS, group, 0)

    pltpu.sync_copy(ids_v, ids_hbm.at[pl.ds(base, _E)])
    pltpu.sync_copy(mask_v, mask_hbm.at[pl.ds(base, _E)])


def kernel(int_feats, col_offsets, id_bases):
    feats_flat = int_feats.reshape(-1)
    p = jnp.arange(_PERIOD, dtype=jnp.int32)
    jcol = p % _F
    pat_idx = (p - jcol + col_offsets[jcol]).reshape(_KVECS, _LANES)
    pat_base = id_bases[jcol].reshape(_KVECS, _LANES)
    ids_flat, mask_flat = _tokenize(feats_flat, pat_idx, pat_base)
    virtual_ids = ids_flat.reshape(_BATCH, _F)
    valid_mask = mask_flat.reshape(_BATCH, _F).astype(jnp.bool_)
    return virtual_ids, valid_mask


# speedup vs baseline: 1.2805x; 1.2805x over previous
import functools

import jax
import jax.numpy as jnp
from jax import lax
from jax.experimental import pallas as pl
from jax.experimental.pallas import tpu as pltpu
from jax.experimental.pallas import tpu_sc as plsc

_F = 26
_NB = 10000
_BATCH = 16384
_LANES = 16
_PERIOD = 208
_KVECS = _PERIOD // _LANES
_NC = 2
_NS = 16
_NW = _NC * _NS
_E = _BATCH * _F // _NW
_GROUPS = _E // _PERIOD


@functools.partial(
    pl.kernel,
    mesh=plsc.VectorSubcoreMesh(core_axis_name="c", subcore_axis_name="s"),
    out_type=[
        jax.ShapeDtypeStruct((_BATCH * _F,), jnp.int32),
        jax.ShapeDtypeStruct((_BATCH * _F,), jnp.int32),
    ],
    scratch_types=[
        pltpu.VMEM((_E,), jnp.int32),
        pltpu.VMEM((_E,), jnp.int32),
        pltpu.VMEM((_E,), jnp.int32),
        pltpu.VMEM((_KVECS, _LANES), jnp.int32),
        pltpu.VMEM((_KVECS, _LANES), jnp.int32),
    ],
    compiler_params=pltpu.CompilerParams(needs_layout_passes=False),
)
def _tokenize(feats_hbm, pidx_hbm, pbase_hbm, ids_hbm, mask_hbm,
              feats_v, ids_v, mask_v, pidx_v, pbase_v):
    wid = lax.axis_index("s") * _NC + lax.axis_index("c")
    base = wid * _E
    pltpu.sync_copy(feats_hbm.at[pl.ds(base, _E)], feats_v)
    pltpu.sync_copy(pidx_hbm, pidx_v)
    pltpu.sync_copy(pbase_hbm, pbase_v)

    pidx = [pidx_v[kk, :] for kk in range(_KVECS)]
    pbase = [pbase_v[kk, :] for kk in range(_KVECS)]

    @plsc.parallel_loop(0, _GROUPS, 1, unroll=2)
    def group(g):
        goff = g * _PERIOD
        for kk in range(_KVECS):
            vals = plsc.load_gather(feats_v, [pidx[kk] + goff])
            r = vals
            for c in (8 * _NB, 4 * _NB, 2 * _NB, _NB):
                r = jnp.where(r >= c, r - c, r)
            vid = pbase[kk] + r + 1
            valid = vals > 0
            off = goff + kk * _LANES
            ids_v[pl.ds(off, _LANES)] = jnp.where(valid, vid, 0)
            mask_v[pl.ds(off, _LANES)] = jnp.where(valid, 1, 0)

    pltpu.sync_copy(ids_v, ids_hbm.at[pl.ds(base, _E)])
    pltpu.sync_copy(mask_v, mask_hbm.at[pl.ds(base, _E)])


def kernel(int_feats, col_offsets, id_bases):
    feats_flat = int_feats.reshape(-1)
    p = jnp.arange(_PERIOD, dtype=jnp.int32)
    jcol = p % _F
    pat_idx = (p - jcol + col_offsets[jcol]).reshape(_KVECS, _LANES)
    pat_base = id_bases[jcol].reshape(_KVECS, _LANES)
    ids_flat, mask_flat = _tokenize(feats_flat, pat_idx, pat_base)
    virtual_ids = ids_flat.reshape(_BATCH, _F)
    valid_mask = mask_flat.reshape(_BATCH, _F).astype(jnp.bool_)
    return virtual_ids, valid_mask


# speedup vs baseline: 3.8808x; 3.0306x over previous
import functools

import jax
import jax.numpy as jnp
from jax.experimental import pallas as pl
from jax.experimental.pallas import tpu as pltpu

_F = 26
_NB = 10000
_BATCH = 16384
_BS = 2048


def _body(feats_ref, perm_ref, base_ref, ids_ref, mask_ref):
    x = feats_ref[...].astype(jnp.float32)
    raw = jnp.dot(x, perm_ref[...], preferred_element_type=jnp.float32,
                  precision=jax.lax.Precision.HIGHEST)
    q = jnp.floor(raw * (1.0 / _NB))
    r = raw - q * _NB
    r = jnp.where(r >= _NB, r - _NB, r)
    r = jnp.where(r < 0, r + _NB, r)
    valid = raw > 0
    vid = jnp.where(valid, base_ref[0:1, :] + r + 1.0, 0.0)
    ids_ref[...] = vid.astype(jnp.int32)
    mask_ref[...] = valid


@functools.partial(jax.jit, static_argnums=())
def _tokenize(int_feats, perm, base_rows):
    grid = (_BATCH // _BS,)
    return pl.pallas_call(
        _body,
        grid=grid,
        in_specs=[
            pl.BlockSpec((_BS, _F), lambda i: (i, 0)),
            pl.BlockSpec((_F, _F), lambda i: (0, 0)),
            pl.BlockSpec((8, _F), lambda i: (0, 0)),
        ],
        out_specs=[
            pl.BlockSpec((_BS, _F), lambda i: (i, 0)),
            pl.BlockSpec((_BS, _F), lambda i: (i, 0)),
        ],
        out_shape=[
            jax.ShapeDtypeStruct((_BATCH, _F), jnp.int32),
            jax.ShapeDtypeStruct((_BATCH, _F), jnp.bool_),
        ],
        compiler_params=pltpu.CompilerParams(
            dimension_semantics=("arbitrary",),
        ),
    )(int_feats, perm, base_rows)


def kernel(int_feats, col_offsets, id_bases):
    perm = (jnp.arange(_F, dtype=jnp.int32)[:, None]
            == col_offsets[None, :]).astype(jnp.float32)
    base_rows = jnp.broadcast_to(id_bases.astype(jnp.float32)[None, :], (8, _F))
    virtual_ids, valid_mask = _tokenize(int_feats, perm, base_rows)
    return virtual_ids, valid_mask
